# baseline (speedup 1.0000x reference)
import functools

import jax
import jax.numpy as jnp
from jax import lax
from jax.experimental import pallas as pl
from jax.experimental.pallas import tpu as pltpu
from jax.experimental.pallas import tpu_sc as plsc

NUM_HEAD = 100
EMBED_DIM = 256
BATCH = 16384

_info = plsc.get_sparse_core_info()
_NC, _NS, _NL = _info.num_cores, _info.num_subcores, _info.num_lanes
_NW = _NC * _NS
_B_PER_W = BATCH // _NW
_CHUNK = 128
_NPHASE = 12
_PROWS = 8

_mesh = plsc.VectorSubcoreMesh(core_axis_name="c", subcore_axis_name="s")


@functools.partial(
    pl.kernel,
    mesh=_mesh,
    out_type=jax.ShapeDtypeStruct((BATCH, 1, EMBED_DIM), jnp.float32),
    scratch_types=[
        pltpu.VMEM((NUM_HEAD, EMBED_DIM), jnp.float32),
        pltpu.VMEM((_B_PER_W,), jnp.int32),
        pltpu.VMEM((_CHUNK, EMBED_DIM), jnp.float32),
        pltpu.SemaphoreType.DMA,
        pltpu.SemaphoreType.DMA,
    ],
)
def _gather_kernel(table_hbm, idx_hbm, out_hbm, table_v, idx_v, dummy_v, sem, stg_sem):
    wid = lax.axis_index("s") * _NC + lax.axis_index("c")
    base = wid * _B_PER_W

    pltpu.sync_copy(idx_hbm.at[pl.ds(base, _B_PER_W)], idx_v)

    stg = []
    for j in range(_NPHASE):
        p = lax.rem(wid + j, _NPHASE)
        row = pl.multiple_of(p * _PROWS, _PROWS)
        stg.append(
            pltpu.async_copy(
                table_hbm.at[pl.ds(row, _PROWS)],
                table_v.at[pl.ds(row, _PROWS)],
                stg_sem,
            )
        )
    tail = _NPHASE * _PROWS
    stg.append(
        pltpu.async_copy(
            table_hbm.at[pl.ds(tail, NUM_HEAD - tail)],
            table_v.at[pl.ds(tail, NUM_HEAD - tail)],
            stg_sem,
        )
    )
    for h in stg:
        h.wait()

    def body(g, _):
        lblv = idx_v[pl.ds(g * _NL, _NL)]
        for k in range(_NL):
            pltpu.async_copy(
                table_v.at[lblv[k]],
                out_hbm.at[base + g * _NL + k, 0],
                sem,
            )
        return 0

    lax.fori_loop(0, _B_PER_W // _NL, body, 0)
    for i in range(_B_PER_W // _CHUNK):
        pltpu.make_async_copy(
            out_hbm.at[pl.ds(base + i * _CHUNK, _CHUNK), 0], dummy_v, sem
        ).wait()


def kernel(params, labels, head_table):
    del params
    return _gather_kernel(head_table, labels)


# speedup vs baseline: 1.1777x; 1.1777x over previous
import functools

import jax
import jax.numpy as jnp
from jax import lax
from jax.experimental import pallas as pl
from jax.experimental.pallas import tpu as pltpu
from jax.experimental.pallas import tpu_sc as plsc

NUM_HEAD = 100
EMBED_DIM = 256
BATCH = 16384

_info = plsc.get_sparse_core_info()
_NC, _NS = _info.num_cores, _info.num_subcores
_NW = _NC * _NS
_B_PER_W = BATCH // _NW
_CHUNK = 128

_mesh = plsc.VectorSubcoreMesh(core_axis_name="c", subcore_axis_name="s")


@functools.partial(
    pl.kernel,
    mesh=_mesh,
    out_type=jax.ShapeDtypeStruct((BATCH, 1, EMBED_DIM), jnp.float32),
    scratch_types=[
        pltpu.VMEM((NUM_HEAD, EMBED_DIM), jnp.float32),
        pltpu.VMEM((_B_PER_W,), jnp.int32),
        pltpu.VMEM((_CHUNK, EMBED_DIM), jnp.float32),
        pltpu.SemaphoreType.DMA,
    ],
)
def _gather_kernel(table_hbm, idx_hbm, out_hbm, table_v, idx_v, dummy_v, sem):
    cidx = lax.axis_index("c")
    sidx = lax.axis_index("s")
    wid = sidx * _NC + cidx
    base = wid * _B_PER_W

    pltpu.sync_copy(idx_hbm.at[pl.ds(base, _B_PER_W)], idx_v)
    pltpu.sync_copy(table_hbm.at[cidx * (_NS // 2) + sidx // 2], table_v)

    _NL = 16

    def body(g, _):
        lblv = idx_v[pl.ds(g * _NL, _NL)]
        for k in range(_NL):
            pltpu.async_copy(
                table_v.at[lblv[k]],
                out_hbm.at[base + g * _NL + k, 0],
                sem,
            )
        return 0

    lax.fori_loop(0, _B_PER_W // _NL, body, 0)
    for i in range(_B_PER_W // _CHUNK):
        pltpu.make_async_copy(
            out_hbm.at[pl.ds(base + i * _CHUNK, _CHUNK), 0], dummy_v, sem
        ).wait()


def kernel(params, labels, head_table):
    del params
    table_rep = jnp.broadcast_to(head_table[None], (_NW // 2, NUM_HEAD, EMBED_DIM))
    return _gather_kernel(table_rep, labels)
